# Initial kernel scaffold; baseline (speedup 1.0000x reference)
import jax
import jax.numpy as jnp
from jax.experimental import pallas as pl


def kernel(img):
    raise NotImplementedError("write your pallas kernel here")


# speedup vs baseline: 2.7170x; 2.7170x over previous
import jax
import jax.numpy as jnp
from jax.experimental import pallas as pl

B, C, H, W = 8, 3, 512, 512
PATCH = 16
GY, GX = H // PATCH, W // PATCH
NSEG_PER_IMG = GY * GX
NV = B * NSEG_PER_IMG
ROWS = B * H
HB = 256


def _fused_kernel(img_ref, fv_ref, seg_ref, byx_ref, bbox_ref):
    i = pl.program_id(0)
    x = img_ref[0]

    fv_ref[:, 0 * W:1 * W] = x[0]
    fv_ref[:, 1 * W:2 * W] = x[1]
    fv_ref[:, 2 * W:3 * W] = x[2]

    row0 = i * HB
    rloc = jax.lax.broadcasted_iota(jnp.int32, (HB, W), 0)
    lane = jax.lax.broadcasted_iota(jnp.int32, (HB, W), 1)
    r = rloc + row0
    bb = r // H
    yy = r % H
    seg_ref[...] = bb * NSEG_PER_IMG + (yy // PATCH) * GX + (lane // PATCH)
    byx_ref[...] = jnp.stack([bb, yy, lane], axis=0)

    @pl.when(i == 0)
    def _():
        j = jax.lax.broadcasted_iota(jnp.int32, (4, NV), 0)
        v = jax.lax.broadcasted_iota(jnp.int32, (4, NV), 1)
        off = jax.lax.broadcasted_iota(jnp.int32, (PATCH, PATCH), 0)
        omin = jnp.min(off)
        omax = jnp.max(off)
        py = (v % NSEG_PER_IMG) // GX
        px = v % GX
        ymin = py * PATCH + omin
        ymax = py * PATCH + omax
        xmin = px * PATCH + omin
        xmax = px * PATCH + omax
        bbox_ref[...] = jnp.where(
            j == 0, ymin, jnp.where(j == 1, xmin, jnp.where(j == 2, ymax, xmax))
        )


def kernel(img):
    grid = (ROWS // HB,)
    blocks_per_img = H // HB

    fv2, seg2, byx3, bbox = pl.pallas_call(
        _fused_kernel,
        grid=grid,
        in_specs=[
            pl.BlockSpec((1, C, HB, W),
                         lambda i: (i // blocks_per_img, 0, i % blocks_per_img, 0)),
        ],
        out_specs=[
            pl.BlockSpec((HB, W * C), lambda i: (i, 0)),
            pl.BlockSpec((HB, W), lambda i: (i, 0)),
            pl.BlockSpec((3, HB, W), lambda i: (0, i, 0)),
            pl.BlockSpec((4, NV), lambda i: (0, 0)),
        ],
        out_shape=[
            jax.ShapeDtypeStruct((ROWS, W * C), jnp.float32),
            jax.ShapeDtypeStruct((ROWS, W), jnp.int32),
            jax.ShapeDtypeStruct((3, ROWS, W), jnp.int32),
            jax.ShapeDtypeStruct((4, NV), jnp.int32),
        ],
    )(img)

    fV = fv2.reshape(B * H * W, C)
    seg = seg2.reshape(B * H * W)
    byx = byx3.reshape(3, B * H * W)
    return (fV, seg, byx, bbox)


# speedup vs baseline: 3.2220x; 1.1859x over previous
import jax
import jax.numpy as jnp
from jax.experimental import pallas as pl

B, C, H, W = 8, 3, 512, 512
PATCH = 16
GY, GX = H // PATCH, W // PATCH
NSEG_PER_IMG = GY * GX
NV = B * NSEG_PER_IMG
N = B * H * W
HB = 16
PIXB = HB * W


def _fused_kernel(img_ref, fv_ref, seg_ref, byx_ref, bbox_ref):
    i = pl.program_id(0)
    x = img_ref[0]

    fv_ref[...] = jnp.transpose(x, (1, 2, 0)).reshape(PIXB, C)

    n0 = i * PIXB
    j = jax.lax.broadcasted_iota(jnp.int32, (3, PIXB), 0)
    n = jax.lax.broadcasted_iota(jnp.int32, (3, PIXB), 1) + n0
    bb = n // (H * W)
    rem = n % (H * W)
    yy = rem // W
    xx = rem % W
    byx_ref[...] = jnp.where(j == 0, bb, jnp.where(j == 1, yy, xx))
    seg_ref[...] = (bb * NSEG_PER_IMG + (yy // PATCH) * GX + xx // PATCH)[0]

    @pl.when(i == 0)
    def _():
        jb = jax.lax.broadcasted_iota(jnp.int32, (4, NV), 0)
        v = jax.lax.broadcasted_iota(jnp.int32, (4, NV), 1)
        off = jax.lax.broadcasted_iota(jnp.int32, (PATCH, PATCH), 0)
        omin = jnp.min(off)
        omax = jnp.max(off)
        py = (v % NSEG_PER_IMG) // GX
        px = v % GX
        bbox_ref[...] = jnp.where(
            jb == 0, py * PATCH + omin,
            jnp.where(jb == 1, px * PATCH + omin,
                      jnp.where(jb == 2, py * PATCH + omax,
                                px * PATCH + omax)))


def kernel(img):
    grid = (N // PIXB,)
    blocks_per_img = H // HB

    fV, seg, byx, bbox = pl.pallas_call(
        _fused_kernel,
        grid=grid,
        in_specs=[
            pl.BlockSpec((1, C, HB, W),
                         lambda i: (i // blocks_per_img, 0, i % blocks_per_img, 0)),
        ],
        out_specs=[
            pl.BlockSpec((PIXB, C), lambda i: (i, 0)),
            pl.BlockSpec((PIXB,), lambda i: (i,)),
            pl.BlockSpec((3, PIXB), lambda i: (0, i)),
            pl.BlockSpec((4, NV), lambda i: (0, 0)),
        ],
        out_shape=[
            jax.ShapeDtypeStruct((N, C), jnp.float32),
            jax.ShapeDtypeStruct((N,), jnp.int32),
            jax.ShapeDtypeStruct((3, N), jnp.int32),
            jax.ShapeDtypeStruct((4, NV), jnp.int32),
        ],
    )(img)
    return (fV, seg, byx, bbox)


# speedup vs baseline: 3.5669x; 1.1070x over previous
import jax
import jax.numpy as jnp
from jax.experimental import pallas as pl

B, C, H, W = 8, 3, 512, 512
PATCH = 16
GY, GX = H // PATCH, W // PATCH
NSEG_PER_IMG = GY * GX
NV = B * NSEG_PER_IMG
N = B * H * W
HB = 64
PIXB = HB * W
CHUNK = 131072


def _fv_kernel(img_ref, fv_ref):
    x = img_ref[0]
    fv_ref[...] = jnp.transpose(x, (1, 2, 0)).reshape(PIXB, C)


def _idx_kernel(seg_ref, byx_ref, bbox_ref):
    i = pl.program_id(0)
    j = jax.lax.broadcasted_iota(jnp.int32, (3, CHUNK), 0)
    n = jax.lax.broadcasted_iota(jnp.int32, (3, CHUNK), 1) + i * CHUNK
    bb = n // (H * W)
    rem = n % (H * W)
    yy = rem // W
    xx = rem % W
    byx_ref[...] = jnp.where(j == 0, bb, jnp.where(j == 1, yy, xx))
    seg_ref[...] = (bb * NSEG_PER_IMG + (yy // PATCH) * GX + xx // PATCH)[0]

    @pl.when(i == 0)
    def _():
        jb = jax.lax.broadcasted_iota(jnp.int32, (4, NV), 0)
        v = jax.lax.broadcasted_iota(jnp.int32, (4, NV), 1)
        off = jax.lax.broadcasted_iota(jnp.int32, (PATCH, PATCH), 0)
        omin = jnp.min(off)
        omax = jnp.max(off)
        py = (v % NSEG_PER_IMG) // GX
        px = v % GX
        bbox_ref[...] = jnp.where(
            jb == 0, py * PATCH + omin,
            jnp.where(jb == 1, px * PATCH + omin,
                      jnp.where(jb == 2, py * PATCH + omax,
                                px * PATCH + omax)))


def kernel(img):
    blocks_per_img = H // HB
    fV = pl.pallas_call(
        _fv_kernel,
        grid=(N // PIXB,),
        in_specs=[
            pl.BlockSpec((1, C, HB, W),
                         lambda i: (i // blocks_per_img, 0, i % blocks_per_img, 0)),
        ],
        out_specs=pl.BlockSpec((PIXB, C), lambda i: (i, 0)),
        out_shape=jax.ShapeDtypeStruct((N, C), jnp.float32),
    )(img)

    seg, byx, bbox = pl.pallas_call(
        _idx_kernel,
        grid=(N // CHUNK,),
        in_specs=[],
        out_specs=[
            pl.BlockSpec((CHUNK,), lambda i: (i,)),
            pl.BlockSpec((3, CHUNK), lambda i: (0, i)),
            pl.BlockSpec((4, NV), lambda i: (0, 0)),
        ],
        out_shape=[
            jax.ShapeDtypeStruct((N,), jnp.int32),
            jax.ShapeDtypeStruct((3, N), jnp.int32),
            jax.ShapeDtypeStruct((4, NV), jnp.int32),
        ],
    )()
    return (fV, seg, byx, bbox)
